# baseline (speedup 1.0000x reference)
import functools

import jax
import jax.numpy as jnp
from jax.experimental import pallas as pl
from jax.experimental.pallas import tpu as pltpu


_N_PROJ = 8192
_N_PC = 8192
_D = 128
_DA = _D + 2

_BI = 512
_BJ = 2048
_NI = _N_PROJ // _BI
_NJ = _N_PC // _BJ


def _dist_loss_kernel(a_ref, bt_ref, aa_ref, pe_ref, mp_ref,
                      ft_ref, mp_out_ref, minacc_ref):
    j = pl.program_id(1)

    ab = jax.lax.dot_general(
        a_ref[...], bt_ref[...],
        dimension_numbers=(((1,), (0,)), ((), ())),
        preferred_element_type=jnp.float32)
    pm = jnp.min(ab, axis=1, keepdims=True)

    @pl.when(j == 0)
    def _():
        minacc_ref[...] = pm

    @pl.when(j > 0)
    def _():
        minacc_ref[...] = jnp.minimum(minacc_ref[...], pm)

    @pl.when(j == _NJ - 1)
    def _():
        d = minacc_ref[...] + aa_ref[...]
        ft = jnp.abs(jnp.sqrt(jnp.abs(d) + 1e-7) - jnp.abs(pe_ref[...]))
        ft_ref[...] = ft
        mp_out_ref[...] = jnp.abs(mp_ref[...])


@functools.partial(jax.jit, static_argnames=("interpret",))
def _dist_loss(a_aug, bt_aug, aa, proj_eval, manifold, interpret=False):
    ft, mp_abs = pl.pallas_call(
        _dist_loss_kernel,
        grid=(_NI, _NJ),
        in_specs=[
            pl.BlockSpec((_BI, _DA), lambda i, j: (i, 0)),
            pl.BlockSpec((_DA, _BJ), lambda i, j: (0, j)),
            pl.BlockSpec((_BI, 1), lambda i, j: (i, 0)),
            pl.BlockSpec((_BI, 1), lambda i, j: (i, 0)),
            pl.BlockSpec((_BI, 1), lambda i, j: (i, 0)),
        ],
        out_specs=[
            pl.BlockSpec((_BI, 1), lambda i, j: (i, 0)),
            pl.BlockSpec((_BI, 1), lambda i, j: (i, 0)),
        ],
        out_shape=[
            jax.ShapeDtypeStruct((_N_PROJ, 1), jnp.float32),
            jax.ShapeDtypeStruct((_N_PROJ, 1), jnp.float32),
        ],
        scratch_shapes=[pltpu.VMEM((_BI, 1), jnp.float32)],
        compiler_params=pltpu.CompilerParams(
            dimension_semantics=("parallel", "arbitrary"),
        ),
        interpret=interpret,
    )(a_aug, bt_aug, aa, proj_eval, manifold)
    return ft, mp_abs


def kernel(zerolevelset_points, genlevelset_points, pc_input,
           zerolevelset_eval, gen_points_eval, manifold_pnts_pred,
           loss_lambda):
    if zerolevelset_points is not None:
        proj = jnp.concatenate([zerolevelset_points, genlevelset_points], axis=0)
        proj_eval = jnp.concatenate([zerolevelset_eval, gen_points_eval], axis=0)
    else:
        proj = genlevelset_points
        proj_eval = gen_points_eval

    n = proj.shape[0]
    aa = jnp.sum(proj * proj, axis=1, keepdims=True)
    ones = jnp.ones((n, 2), jnp.bfloat16)
    a_aug = jnp.concatenate(
        [(-2.0 * proj).astype(jnp.bfloat16), ones], axis=1)

    bb = jnp.sum(pc_input * pc_input, axis=1)
    bb_hi = bb.astype(jnp.bfloat16)
    bb_lo = (bb - bb_hi.astype(jnp.float32)).astype(jnp.bfloat16)
    bt_aug = jnp.concatenate(
        [pc_input.T.astype(jnp.bfloat16),
         bb_hi[None, :], bb_lo[None, :]], axis=0)

    ft, mp_abs = _dist_loss(a_aug, bt_aug, aa, proj_eval, manifold_pnts_pred)
    mean_first = jnp.mean(ft)
    mean_second = jnp.mean(mp_abs)
    ll = 0.1 if loss_lambda is None else loss_lambda
    loss = mean_first + ll * mean_second
    return (loss, mean_first, mean_second)


# speedup vs baseline: 1.3473x; 1.3473x over previous
import functools

import jax
import jax.numpy as jnp
from jax.experimental import pallas as pl
from jax.experimental.pallas import tpu as pltpu


_N_PROJ = 8192
_N_PC = 8192
_D = 128
_DA = _D + 2

_BI = 512
_BJ = 2048
_NI = _N_PROJ // _BI
_NJ = _N_PC // _BJ


def _dist_loss_kernel(a_ref, bt_ref, aa_ref, pe_ref, mp_ref,
                      ft_ref, mp_out_ref):
    pm = None
    for j in range(_NJ):
        ab = jax.lax.dot_general(
            a_ref[...], bt_ref[:, j * _BJ:(j + 1) * _BJ],
            dimension_numbers=(((1,), (0,)), ((), ())),
            preferred_element_type=jnp.float32)
        m = jnp.min(ab, axis=1, keepdims=True)
        pm = m if pm is None else jnp.minimum(pm, m)

    d = pm + aa_ref[...]
    ft = jnp.abs(jnp.sqrt(jnp.abs(d) + 1e-7) - jnp.abs(pe_ref[...]))
    ft_ref[...] = ft
    mp_out_ref[...] = jnp.abs(mp_ref[...])


@functools.partial(jax.jit, static_argnames=("interpret",))
def _dist_loss(a_aug, bt_aug, aa, proj_eval, manifold, interpret=False):
    ft, mp_abs = pl.pallas_call(
        _dist_loss_kernel,
        grid=(_NI,),
        in_specs=[
            pl.BlockSpec((_BI, _DA), lambda i: (i, 0)),
            pl.BlockSpec((_DA, _N_PC), lambda i: (0, 0)),
            pl.BlockSpec((_BI, 1), lambda i: (i, 0)),
            pl.BlockSpec((_BI, 1), lambda i: (i, 0)),
            pl.BlockSpec((_BI, 1), lambda i: (i, 0)),
        ],
        out_specs=[
            pl.BlockSpec((_BI, 1), lambda i: (i, 0)),
            pl.BlockSpec((_BI, 1), lambda i: (i, 0)),
        ],
        out_shape=[
            jax.ShapeDtypeStruct((_N_PROJ, 1), jnp.float32),
            jax.ShapeDtypeStruct((_N_PROJ, 1), jnp.float32),
        ],
        compiler_params=pltpu.CompilerParams(
            dimension_semantics=("arbitrary",),
        ),
        interpret=interpret,
    )(a_aug, bt_aug, aa, proj_eval, manifold)
    return ft, mp_abs


def kernel(zerolevelset_points, genlevelset_points, pc_input,
           zerolevelset_eval, gen_points_eval, manifold_pnts_pred,
           loss_lambda):
    if zerolevelset_points is not None:
        proj = jnp.concatenate([zerolevelset_points, genlevelset_points], axis=0)
        proj_eval = jnp.concatenate([zerolevelset_eval, gen_points_eval], axis=0)
    else:
        proj = genlevelset_points
        proj_eval = gen_points_eval

    n = proj.shape[0]
    aa = jnp.sum(proj * proj, axis=1, keepdims=True)
    ones = jnp.ones((n, 2), jnp.bfloat16)
    a_aug = jnp.concatenate(
        [(-2.0 * proj).astype(jnp.bfloat16), ones], axis=1)

    bb = jnp.sum(pc_input * pc_input, axis=1)
    bb_hi = bb.astype(jnp.bfloat16)
    bb_lo = (bb - bb_hi.astype(jnp.float32)).astype(jnp.bfloat16)
    bt_aug = jnp.concatenate(
        [pc_input.T.astype(jnp.bfloat16),
         bb_hi[None, :], bb_lo[None, :]], axis=0)

    ft, mp_abs = _dist_loss(a_aug, bt_aug, aa, proj_eval, manifold_pnts_pred)
    mean_first = jnp.mean(ft)
    mean_second = jnp.mean(mp_abs)
    ll = 0.1 if loss_lambda is None else loss_lambda
    loss = mean_first + ll * mean_second
    return (loss, mean_first, mean_second)


# speedup vs baseline: 1.4539x; 1.0791x over previous
import functools

import jax
import jax.numpy as jnp
from jax.experimental import pallas as pl
from jax.experimental.pallas import tpu as pltpu


_N_HALF = 4096
_N_PROJ = 8192
_N_PC = 8192
_D = 128
_DA = _D + 2

_BI = 512
_BJ = 2048
_NI = _N_PROJ // _BI
_NJ = _N_PC // _BJ
_NI_HALF = _N_HALF // _BI


def _dist_loss_kernel(z_ref, g_ref, bt_ref, ze_ref, ge_ref, mp_ref,
                      ft_sum_ref, mp_sum_ref, a_scr):
    i = pl.program_id(0)
    first_half = i < _NI_HALF

    a = jnp.where(first_half, z_ref[...], g_ref[...])
    aa = jnp.sum(a * a, axis=1, keepdims=True)

    @pl.when(i == 0)
    def _():
        a_scr[:, _D:] = jnp.ones((_BI, 2), jnp.bfloat16)

    a_scr[:, :_D] = (-2.0 * a).astype(jnp.bfloat16)
    a_aug = a_scr[...]

    pm = None
    for j in range(_NJ):
        ab = jax.lax.dot_general(
            a_aug, bt_ref[:, j * _BJ:(j + 1) * _BJ],
            dimension_numbers=(((1,), (0,)), ((), ())),
            preferred_element_type=jnp.float32)
        m = jnp.min(ab, axis=1, keepdims=True)
        pm = m if pm is None else jnp.minimum(pm, m)

    d = pm + aa
    pe = jnp.where(first_half, ze_ref[...], ge_ref[...])
    ft = jnp.abs(jnp.sqrt(jnp.abs(d) + 1e-7) - jnp.abs(pe))
    ft_blk = jnp.sum(ft)
    mp_blk = jnp.sum(jnp.abs(mp_ref[...]))

    @pl.when(i == 0)
    def _():
        ft_sum_ref[0, 0] = ft_blk
        mp_sum_ref[0, 0] = mp_blk

    @pl.when(i > 0)
    def _():
        ft_sum_ref[0, 0] += ft_blk
        mp_sum_ref[0, 0] += mp_blk


def _half_map(i):
    return (jnp.minimum(i, _NI_HALF - 1), 0)


def _gen_map(i):
    return (jnp.maximum(i - _NI_HALF, 0), 0)


@functools.partial(jax.jit, static_argnames=("interpret",))
def _dist_loss(zero_pts, gen_pts, bt_aug, zero_eval, gen_eval, manifold,
               interpret=False):
    ft_sum, mp_sum = pl.pallas_call(
        _dist_loss_kernel,
        grid=(_NI,),
        in_specs=[
            pl.BlockSpec((_BI, _D), _half_map),
            pl.BlockSpec((_BI, _D), _gen_map),
            pl.BlockSpec((_DA, _N_PC), lambda i: (0, 0)),
            pl.BlockSpec((_BI, 1), _half_map),
            pl.BlockSpec((_BI, 1), _gen_map),
            pl.BlockSpec((_BI, 1), lambda i: (i, 0)),
        ],
        out_specs=[
            pl.BlockSpec(memory_space=pltpu.SMEM),
            pl.BlockSpec(memory_space=pltpu.SMEM),
        ],
        out_shape=[
            jax.ShapeDtypeStruct((1, 1), jnp.float32),
            jax.ShapeDtypeStruct((1, 1), jnp.float32),
        ],
        scratch_shapes=[pltpu.VMEM((_BI, _DA), jnp.bfloat16)],
        compiler_params=pltpu.CompilerParams(
            dimension_semantics=("arbitrary",),
        ),
        interpret=interpret,
    )(zero_pts, gen_pts, bt_aug, zero_eval, gen_eval, manifold)
    return ft_sum[0, 0], mp_sum[0, 0]


def kernel(zerolevelset_points, genlevelset_points, pc_input,
           zerolevelset_eval, gen_points_eval, manifold_pnts_pred,
           loss_lambda):
    bb = jnp.sum(pc_input * pc_input, axis=1)
    bb_hi = bb.astype(jnp.bfloat16)
    bb_lo = (bb - bb_hi.astype(jnp.float32)).astype(jnp.bfloat16)
    bt_aug = jnp.concatenate(
        [pc_input.astype(jnp.bfloat16).T,
         bb_hi[None, :], bb_lo[None, :]], axis=0)

    ft_sum, mp_sum = _dist_loss(
        zerolevelset_points, genlevelset_points, bt_aug,
        zerolevelset_eval, gen_points_eval, manifold_pnts_pred)

    mean_first = ft_sum / _N_PROJ
    mean_second = mp_sum / _N_PROJ
    ll = 0.1 if loss_lambda is None else loss_lambda
    loss = mean_first + ll * mean_second
    return (loss, mean_first, mean_second)
